# baseline (speedup 1.0000x reference)
import jax
import jax.numpy as jnp
from jax import lax
from jax.experimental import pallas as pl
from jax.experimental.pallas import tpu as pltpu
from jax.experimental.pallas import tpu_sc as plsc

VOCAB = 256
D_MODEL = 384

_NC = 2
_NS = 16
_NW = _NC * _NS

_N = 4 * 8192
_BPW = _N // _NW
_CH = 128
_NCH = _BPW // _CH
_NBUF = 2


def _table_body(e_ref, w_ref, b_ref, t_ref):
    t_ref[...] = lax.dot_general(
        e_ref[...], w_ref[...],
        dimension_numbers=(((1,), (1,)), ((), ())),
        preferred_element_type=jnp.float32,
    ) + b_ref[...]


_fuse_table = pl.pallas_call(
    _table_body,
    out_shape=jax.ShapeDtypeStruct((VOCAB, D_MODEL), jnp.float32),
)


_ROWS_PER_B = 8192 // _BPW


def _gather_body(ids_hbm, table_hbm, out_hbm, idx_v, rows_v, *sems):
    wid = lax.axis_index("s") * _NC + lax.axis_index("c")
    bb = wid // _ROWS_PER_B
    s0 = (wid % _ROWS_PER_B) * _BPW

    pltpu.sync_copy(ids_hbm.at[bb, pl.ds(s0, _BPW)], idx_v)
    for i in range(_BPW // 16):
        sl = pl.ds(i * 16, 16)
        idx_v[sl] = jnp.clip(idx_v[sl], 0, VOCAB - 1)

    gsems = sems[:_NBUF]
    ssems = sems[_NBUF:]
    gh = [None] * _NBUF
    sh = [None] * _NBUF

    def start_gather(c):
        buf = c % _NBUF
        if sh[buf] is not None:
            sh[buf].wait()
        gh[buf] = pltpu.async_copy(
            table_hbm.at[idx_v.at[pl.ds(c * _CH, _CH)]],
            rows_v.at[buf], gsems[buf])

    for k in range(min(_NBUF - 1, _NCH)):
        start_gather(k)
    for c in range(_NCH):
        buf = c % _NBUF
        gh[buf].wait()
        sh[buf] = pltpu.async_copy(
            rows_v.at[buf],
            out_hbm.at[bb, pl.ds(s0 + c * _CH, _CH)], ssems[buf])
        nxt = c + _NBUF - 1
        if nxt < _NCH:
            start_gather(nxt)
    for buf in range(_NBUF):
        if sh[buf] is not None:
            sh[buf].wait()


_gather = pl.kernel(
    _gather_body,
    out_type=jax.ShapeDtypeStruct((4, 8192, D_MODEL), jnp.float32),
    mesh=plsc.VectorSubcoreMesh(core_axis_name="c", subcore_axis_name="s"),
    scratch_types=[
        pltpu.VMEM((_BPW,), jnp.int32),
        pltpu.VMEM((_NBUF, _CH, D_MODEL), jnp.float32),
    ] + [pltpu.SemaphoreType.DMA] * (2 * _NBUF),
)


@jax.jit
def kernel(byte_ids, embed_table, W, b):
    table = _fuse_table(embed_table, W, b.reshape(1, D_MODEL))
    return _gather(byte_ids, table)


# speedup vs baseline: 1.0290x; 1.0290x over previous
import jax
import jax.numpy as jnp
from jax import lax
from jax.experimental import pallas as pl
from jax.experimental.pallas import tpu as pltpu
from jax.experimental.pallas import tpu_sc as plsc

VOCAB = 256
D_MODEL = 384

_NC = 2
_NS = 16
_NW = _NC * _NS

_N = 4 * 8192
_BPW = _N // _NW
_CH = 128
_NCH = _BPW // _CH
_NBUF = 2


def _table_body(e_ref, w_ref, b_ref, t_ref):
    t_ref[...] = lax.dot_general(
        e_ref[...], w_ref[...],
        dimension_numbers=(((1,), (1,)), ((), ())),
        preferred_element_type=jnp.float32,
    ) + b_ref[...]


_fuse_table = pl.pallas_call(
    _table_body,
    out_shape=jax.ShapeDtypeStruct((VOCAB, D_MODEL), jnp.float32),
)


_ROWS_PER_B = 8192 // _BPW


def _gather_body(ids_hbm, table_hbm, out_hbm, idx_v, rows_v, *sems):
    wid = lax.axis_index("s") * _NC + lax.axis_index("c")
    bb = wid // _ROWS_PER_B
    s0 = (wid % _ROWS_PER_B) * _BPW

    pltpu.sync_copy(ids_hbm.at[pl.ds(wid * _BPW, _BPW)], idx_v)
    for i in range(_BPW // 16):
        sl = pl.ds(i * 16, 16)
        idx_v[sl] = jnp.clip(idx_v[sl], 0, VOCAB - 1)

    gsems = sems[:_NBUF]
    ssems = sems[_NBUF:]
    gh = [None] * _NBUF
    sh = [None] * _NBUF

    def start_gather(c):
        buf = c % _NBUF
        if sh[buf] is not None:
            sh[buf].wait()
        gh[buf] = pltpu.async_copy(
            table_hbm.at[idx_v.at[pl.ds(c * _CH, _CH)]],
            rows_v.at[buf], gsems[buf])

    for k in range(min(_NBUF - 1, _NCH)):
        start_gather(k)
    for c in range(_NCH):
        buf = c % _NBUF
        gh[buf].wait()
        sh[buf] = pltpu.async_copy(
            rows_v.at[buf],
            out_hbm.at[bb, pl.ds(s0 + c * _CH, _CH)], ssems[buf])
        nxt = c + _NBUF - 1
        if nxt < _NCH:
            start_gather(nxt)
    for buf in range(_NBUF):
        if sh[buf] is not None:
            sh[buf].wait()


_gather = pl.kernel(
    _gather_body,
    out_type=jax.ShapeDtypeStruct((4, 8192, D_MODEL), jnp.float32),
    mesh=plsc.VectorSubcoreMesh(core_axis_name="c", subcore_axis_name="s"),
    scratch_types=[
        pltpu.VMEM((_BPW,), jnp.int32),
        pltpu.VMEM((_NBUF, _CH, D_MODEL), jnp.float32),
    ] + [pltpu.SemaphoreType.DMA] * (2 * _NBUF),
)


@jax.jit
def kernel(byte_ids, embed_table, W, b):
    table = _fuse_table(embed_table, W, b.reshape(1, D_MODEL))
    return _gather(byte_ids.reshape(-1), table)
